# Initial kernel scaffold; baseline (speedup 1.0000x reference)
import jax
import jax.numpy as jnp
from jax.experimental import pallas as pl


def kernel(x, edge_index, edge_weight, Wa1_c, wa2_c, Wa1_o, wa2_o, Wc, bc, Wo, bo):
    raise NotImplementedError("write your pallas kernel here")


# speedup vs baseline: 2.3815x; 2.3815x over previous
import jax
import jax.numpy as jnp
from jax import lax
from jax.experimental import pallas as pl
from jax.experimental.pallas import tpu as pltpu
from jax.experimental.pallas import tpu_sc as plsc

N = 10000
L = 8
D = 768
H = 128
E = 160000

CHUNK = 128
NCH = D // CHUNK
NCHT = 2 * NCH
TILES = 16
EPT = E // TILES
EB = 128
NBATCH = 79
E_PAD = TILES * NBATCH * EB
ROWS_MAIN = 624
ROWS_TAIL = N - ROWS_MAIN * TILES

NB = 200
GRID = N // NB


def _att_body(x_ref, w1c_ref, w2c_ref, w1o_ref, w2o_ref, c_ref, o_ref):
    for (w1_ref, w2_ref, out_ref) in ((w1c_ref, w2c_ref, c_ref),
                                      (w1o_ref, w2o_ref, o_ref)):
        w1 = w1_ref[...]
        w2 = w2_ref[...]
        cols = []
        for l in range(L):
            xl = x_ref[:, l, :]
            h = jnp.tanh(jnp.dot(xl, w1, preferred_element_type=jnp.float32))
            cols.append(jnp.dot(h, w2, preferred_element_type=jnp.float32))
        s = jnp.concatenate(cols, axis=1)
        a = jax.nn.softmax(s, axis=1)
        pooled = jnp.zeros((NB, D), jnp.float32)
        for l in range(L):
            pooled = pooled + a[:, l:l + 1] * x_ref[:, l, :]
        for k in range(NCH):
            out_ref[k] = pooled[:, k * CHUNK:(k + 1) * CHUNK]


def _att_pool(x, w1c, w2c, w1o, w2o):
    out_sds = jax.ShapeDtypeStruct((NCH, N, CHUNK), jnp.float32)
    return pl.pallas_call(
        _att_body,
        grid=(GRID,),
        in_specs=[
            pl.BlockSpec((NB, L, D), lambda i: (i, 0, 0)),
            pl.BlockSpec((D, H), lambda i: (0, 0)),
            pl.BlockSpec((H, 1), lambda i: (0, 0)),
            pl.BlockSpec((D, H), lambda i: (0, 0)),
            pl.BlockSpec((H, 1), lambda i: (0, 0)),
        ],
        out_specs=[
            pl.BlockSpec((NCH, NB, CHUNK), lambda i: (0, i, 0)),
            pl.BlockSpec((NCH, NB, CHUNK), lambda i: (0, i, 0)),
        ],
        out_shape=[out_sds, out_sds],
    )(x, w1c, w2c, w1o, w2o)


def _conv_body(h_ref, src_ref, dst_ref, w_ref, z_ref, out_ref,
               acc, src_v, dst_v, w_v, gidx, rows, sem):
    cid = lax.axis_index("c")
    sid = lax.axis_index("s")

    pltpu.sync_copy(src_ref.at[sid], src_v)
    pltpu.sync_copy(dst_ref.at[sid], dst_v)
    pltpu.sync_copy(w_ref.at[sid], w_v)

    def chunk_body(j, carry):
        chunk = cid * NCH + j

        pltpu.sync_copy(z_ref.at[pl.ds(0, ROWS_MAIN)],
                        acc.at[pl.ds(sid * ROWS_MAIN, ROWS_MAIN)])

        @pl.when(sid == TILES - 1)
        def _zero_tail():
            pltpu.sync_copy(z_ref.at[pl.ds(0, ROWS_TAIL)],
                            acc.at[pl.ds(ROWS_MAIN * TILES, ROWS_TAIL)])
        plsc.subcore_barrier()

        def batch_body(b, carry2):
            for q in range(EB // 16):
                gidx[pl.ds(q * 16, 16)] = (
                    src_v[b, pl.ds(q * 16, 16)] + chunk * N)
            pltpu.async_copy(h_ref.at[gidx], rows, sem).wait()

            def group_body(g, carry3):
                wv = w_v[b, pl.ds(g * 16, 16)]
                for r0 in range(16):
                    r = g * 16 + r0
                    wr = wv[r0]
                    for q in range(CHUNK // 16):
                        rows[r, pl.ds(q * 16, 16)] = (
                            rows[r, pl.ds(q * 16, 16)] * wr)
                return carry3
            lax.fori_loop(0, EB // 16, group_body, 0, unroll=False)

            pltpu.sync_copy(rows, acc.at[dst_v.at[b]], add=True)
            return carry2
        lax.fori_loop(0, NBATCH, batch_body, 0, unroll=False)

        plsc.subcore_barrier()
        pltpu.sync_copy(
            acc.at[pl.ds(sid * ROWS_MAIN, ROWS_MAIN)],
            out_ref.at[pl.ds(chunk * N + sid * ROWS_MAIN, ROWS_MAIN)])

        @pl.when(sid == TILES - 1)
        def _read_tail():
            pltpu.sync_copy(
                acc.at[pl.ds(ROWS_MAIN * TILES, ROWS_TAIL)],
                out_ref.at[pl.ds(chunk * N + ROWS_MAIN * TILES, ROWS_TAIL)])
        return carry
    lax.fori_loop(0, NCH, chunk_body, 0, unroll=False)


def _conv(h_flat, src3, dst3, w3, z):
    mesh = plsc.VectorSubcoreMesh(core_axis_name="c", subcore_axis_name="s")
    f = pl.kernel(
        _conv_body,
        out_type=jax.ShapeDtypeStruct((NCHT * N, CHUNK), jnp.float32),
        mesh=mesh,
        scratch_types=[
            pltpu.VMEM_SHARED((N, CHUNK), jnp.float32),
            pltpu.VMEM((NBATCH, EB), jnp.int32),
            pltpu.VMEM((NBATCH, EB), jnp.int32),
            pltpu.VMEM((NBATCH, EB), jnp.float32),
            pltpu.VMEM((EB,), jnp.int32),
            pltpu.VMEM((EB, CHUNK), jnp.float32),
            pltpu.SemaphoreType.DMA,
        ],
    )
    return f(h_flat, src3, dst3, w3, z)


def _head_body(g_ref, c_ref, o_ref, wc_ref, bc_ref, wo_ref, bo_ref,
               ctr_ref, off_ref):
    acc_c = jnp.zeros((NB, H), jnp.float32)
    acc_o = jnp.zeros((NB, H), jnp.float32)
    for k in range(NCH):
        acc_c = acc_c + jnp.dot(g_ref[k] + c_ref[k], wc_ref[k],
                                preferred_element_type=jnp.float32)
        acc_o = acc_o + jnp.dot(g_ref[NCH + k] + o_ref[k], wo_ref[k],
                                preferred_element_type=jnp.float32)
    ctr_ref[...] = acc_c + bc_ref[...]
    off_ref[...] = jnp.maximum(acc_o + bo_ref[...], 0.0)


def _head(g2, c_att, o_att, wc, bc, wo, bo):
    out_sds = jax.ShapeDtypeStruct((N, H), jnp.float32)
    return pl.pallas_call(
        _head_body,
        grid=(GRID,),
        in_specs=[
            pl.BlockSpec((NCHT, NB, CHUNK), lambda i: (0, i, 0)),
            pl.BlockSpec((NCH, NB, CHUNK), lambda i: (0, i, 0)),
            pl.BlockSpec((NCH, NB, CHUNK), lambda i: (0, i, 0)),
            pl.BlockSpec((NCH, CHUNK, H), lambda i: (0, 0, 0)),
            pl.BlockSpec((1, H), lambda i: (0, 0)),
            pl.BlockSpec((NCH, CHUNK, H), lambda i: (0, 0, 0)),
            pl.BlockSpec((1, H), lambda i: (0, 0)),
        ],
        out_specs=[
            pl.BlockSpec((NB, H), lambda i: (i, 0)),
            pl.BlockSpec((NB, H), lambda i: (i, 0)),
        ],
        out_shape=[out_sds, out_sds],
    )(g2, c_att, o_att, wc, bc, wo, bo)


def kernel(x, edge_index, edge_weight, Wa1_c, wa2_c, Wa1_o, wa2_o, Wc, bc, Wo, bo):
    pad = E_PAD - E
    src3 = jnp.concatenate(
        [edge_index[0], jnp.zeros((pad,), jnp.int32)]).reshape(
            TILES, NBATCH, EB)
    dst3 = jnp.concatenate(
        [edge_index[1], jnp.zeros((pad,), jnp.int32)]).reshape(
            TILES, NBATCH, EB)
    w3 = jnp.concatenate(
        [edge_weight, jnp.zeros((pad,), jnp.float32)]).reshape(
            TILES, NBATCH, EB)
    z = jnp.zeros((ROWS_MAIN, CHUNK), jnp.float32)

    c_att, o_att = _att_pool(x, Wa1_c, wa2_c.reshape(H, 1),
                             Wa1_o, wa2_o.reshape(H, 1))

    h0 = jnp.concatenate([c_att, o_att], axis=0).reshape(NCHT * N, CHUNK)
    g1 = _conv(h0, src3, dst3, w3, z)
    g2 = _conv(g1, src3, dst3, w3, z)

    center, offset = _head(g2.reshape(NCHT, N, CHUNK), c_att, o_att,
                           Wc.reshape(NCH, CHUNK, H), bc.reshape(1, H),
                           Wo.reshape(NCH, CHUNK, H), bo.reshape(1, H))
    return (center, offset)


# speedup vs baseline: 3.1073x; 1.3048x over previous
import jax
import jax.numpy as jnp
from jax import lax
from jax.experimental import pallas as pl
from jax.experimental.pallas import tpu as pltpu
from jax.experimental.pallas import tpu_sc as plsc

N = 10000
L = 8
D = 768
H = 128
E = 160000

CHUNK = 128
NCH = D // CHUNK
NCHT = 2 * NCH
TILES = 16
EPT = E // TILES
EB = 128
NBATCH = 79
E_PAD = TILES * NBATCH * EB
ROWS_MAIN = 624
ROWS_TAIL = N - ROWS_MAIN * TILES

NB = 200
GRID = N // NB


def _att_body(x_ref, w1c_ref, w2c_ref, w1o_ref, w2o_ref, c_ref, o_ref):
    w1c = w1c_ref[...]
    w2c = w2c_ref[...]
    w1o = w1o_ref[...]
    w2o = w2o_ref[...]
    cols_c = []
    cols_o = []
    for l in range(L):
        xl = x_ref[:, l, :]
        hc = jnp.tanh(jnp.dot(xl, w1c, preferred_element_type=jnp.float32))
        cols_c.append(jnp.dot(hc, w2c, preferred_element_type=jnp.float32))
        ho = jnp.tanh(jnp.dot(xl, w1o, preferred_element_type=jnp.float32))
        cols_o.append(jnp.dot(ho, w2o, preferred_element_type=jnp.float32))
    a_c = jax.nn.softmax(jnp.concatenate(cols_c, axis=1), axis=1)
    a_o = jax.nn.softmax(jnp.concatenate(cols_o, axis=1), axis=1)
    pc = jnp.zeros((NB, D), jnp.float32)
    po = jnp.zeros((NB, D), jnp.float32)
    for l in range(L):
        xl = x_ref[:, l, :]
        pc = pc + a_c[:, l:l + 1] * xl
        po = po + a_o[:, l:l + 1] * xl
    for k in range(NCH):
        c_ref[k] = pc[:, k * CHUNK:(k + 1) * CHUNK]
        o_ref[k] = po[:, k * CHUNK:(k + 1) * CHUNK]


def _att_pool(x, w1c, w2c, w1o, w2o):
    out_sds = jax.ShapeDtypeStruct((NCH, N, CHUNK), jnp.float32)
    return pl.pallas_call(
        _att_body,
        grid=(GRID,),
        in_specs=[
            pl.BlockSpec((NB, L, D), lambda i: (i, 0, 0)),
            pl.BlockSpec((D, H), lambda i: (0, 0)),
            pl.BlockSpec((H, 1), lambda i: (0, 0)),
            pl.BlockSpec((D, H), lambda i: (0, 0)),
            pl.BlockSpec((H, 1), lambda i: (0, 0)),
        ],
        out_specs=[
            pl.BlockSpec((NCH, NB, CHUNK), lambda i: (0, i, 0)),
            pl.BlockSpec((NCH, NB, CHUNK), lambda i: (0, i, 0)),
        ],
        out_shape=[out_sds, out_sds],
    )(x, w1c, w2c, w1o, w2o)


def _conv_body(h_ref, ed_ref, w_ref, z_ref, out_ref,
               acc, eb0, eb1, gi0, gi1, rows0, rows1, w_v,
               se0, se1, sr0, sr1):
    cid = lax.axis_index("c")
    sid = lax.axis_index("s")

    pltpu.sync_copy(w_ref.at[sid], w_v)

    def gather(gi, rows, sem):
        pltpu.async_copy(h_ref.at[gi], rows, sem)

    def load_e(b, eb, sem):
        pltpu.async_copy(ed_ref.at[sid, b], eb, sem)

    def wait_e(b, eb, sem):
        pltpu.make_async_copy(ed_ref.at[sid, b], eb, sem).wait()

    def wait_r(gi, rows, sem):
        pltpu.make_async_copy(h_ref.at[gi], rows, sem).wait()

    def mk_gidx(gi, eb, chunk):
        for q in range(EB // 16):
            gi[pl.ds(q * 16, 16)] = eb[0, pl.ds(q * 16, 16)] + chunk * N

    def scale_scatter(rows, eb, b):
        def group_body(g, carry):
            wv = w_v[b, pl.ds(g * 16, 16)]
            for r0 in range(16):
                r = g * 16 + r0
                wr = wv[r0]
                for q in range(CHUNK // 16):
                    rows[r, pl.ds(q * 16, 16)] = (
                        rows[r, pl.ds(q * 16, 16)] * wr)
            return carry
        lax.fori_loop(0, EB // 16, group_body, 0, unroll=False)
        pltpu.sync_copy(rows, acc.at[eb.at[1]], add=True)

    def chunk_body(j, carry):
        chunk = cid * NCH + j

        pltpu.sync_copy(z_ref.at[pl.ds(0, ROWS_MAIN)],
                        acc.at[pl.ds(sid * ROWS_MAIN, ROWS_MAIN)])

        @pl.when(sid == TILES - 1)
        def _zero_tail():
            pltpu.sync_copy(z_ref.at[pl.ds(0, ROWS_TAIL)],
                            acc.at[pl.ds(ROWS_MAIN * TILES, ROWS_TAIL)])
        plsc.subcore_barrier()

        load_e(0, eb0, se0)
        wait_e(0, eb0, se0)
        mk_gidx(gi0, eb0, chunk)
        gather(gi0, rows0, sr0)
        load_e(1, eb1, se1)

        def pair_body(p, carry2):
            b = 2 * p
            wait_e(b + 1, eb1, se1)
            mk_gidx(gi1, eb1, chunk)
            gather(gi1, rows1, sr1)
            wait_r(gi0, rows0, sr0)
            scale_scatter(rows0, eb0, b)
            load_e(b + 2, eb0, se0)
            wait_e(b + 2, eb0, se0)
            mk_gidx(gi0, eb0, chunk)
            gather(gi0, rows0, sr0)
            wait_r(gi1, rows1, sr1)
            scale_scatter(rows1, eb1, b + 1)
            load_e(jnp.minimum(b + 3, NBATCH - 1), eb1, se1)
            return carry2
        lax.fori_loop(0, (NBATCH - 1) // 2, pair_body, 0, unroll=False)

        wait_e(NBATCH - 1, eb1, se1)
        wait_r(gi0, rows0, sr0)
        scale_scatter(rows0, eb0, NBATCH - 1)

        plsc.subcore_barrier()
        pltpu.sync_copy(
            acc.at[pl.ds(sid * ROWS_MAIN, ROWS_MAIN)],
            out_ref.at[pl.ds(chunk * N + sid * ROWS_MAIN, ROWS_MAIN)])

        @pl.when(sid == TILES - 1)
        def _read_tail():
            pltpu.sync_copy(
                acc.at[pl.ds(ROWS_MAIN * TILES, ROWS_TAIL)],
                out_ref.at[pl.ds(chunk * N + ROWS_MAIN * TILES, ROWS_TAIL)])
        return carry
    lax.fori_loop(0, NCH, chunk_body, 0, unroll=False)


def _conv(h_flat, ed, w3, z):
    mesh = plsc.VectorSubcoreMesh(core_axis_name="c", subcore_axis_name="s")
    f = pl.kernel(
        _conv_body,
        out_type=jax.ShapeDtypeStruct((NCHT * N, CHUNK), jnp.float32),
        mesh=mesh,
        scratch_types=[
            pltpu.VMEM_SHARED((N, CHUNK), jnp.float32),
            pltpu.VMEM((2, EB), jnp.int32),
            pltpu.VMEM((2, EB), jnp.int32),
            pltpu.VMEM((EB,), jnp.int32),
            pltpu.VMEM((EB,), jnp.int32),
            pltpu.VMEM((EB, CHUNK), jnp.float32),
            pltpu.VMEM((EB, CHUNK), jnp.float32),
            pltpu.VMEM((NBATCH, EB), jnp.float32),
            pltpu.SemaphoreType.DMA,
            pltpu.SemaphoreType.DMA,
            pltpu.SemaphoreType.DMA,
            pltpu.SemaphoreType.DMA,
        ],
    )
    return f(h_flat, ed, w3, z)


def _head_body(g_ref, c_ref, o_ref, wc_ref, bc_ref, wo_ref, bo_ref,
               ctr_ref, off_ref):
    acc_c = jnp.zeros((NB, H), jnp.float32)
    acc_o = jnp.zeros((NB, H), jnp.float32)
    for k in range(NCH):
        acc_c = acc_c + jnp.dot(g_ref[k] + c_ref[k], wc_ref[k],
                                preferred_element_type=jnp.float32)
        acc_o = acc_o + jnp.dot(g_ref[NCH + k] + o_ref[k], wo_ref[k],
                                preferred_element_type=jnp.float32)
    ctr_ref[...] = acc_c + bc_ref[...]
    off_ref[...] = jnp.maximum(acc_o + bo_ref[...], 0.0)


def _head(g2, c_att, o_att, wc, bc, wo, bo):
    out_sds = jax.ShapeDtypeStruct((N, H), jnp.float32)
    return pl.pallas_call(
        _head_body,
        grid=(GRID,),
        in_specs=[
            pl.BlockSpec((NCHT, NB, CHUNK), lambda i: (0, i, 0)),
            pl.BlockSpec((NCH, NB, CHUNK), lambda i: (0, i, 0)),
            pl.BlockSpec((NCH, NB, CHUNK), lambda i: (0, i, 0)),
            pl.BlockSpec((NCH, CHUNK, H), lambda i: (0, 0, 0)),
            pl.BlockSpec((1, H), lambda i: (0, 0)),
            pl.BlockSpec((NCH, CHUNK, H), lambda i: (0, 0, 0)),
            pl.BlockSpec((1, H), lambda i: (0, 0)),
        ],
        out_specs=[
            pl.BlockSpec((NB, H), lambda i: (i, 0)),
            pl.BlockSpec((NB, H), lambda i: (i, 0)),
        ],
        out_shape=[out_sds, out_sds],
    )(g2, c_att, o_att, wc, bc, wo, bo)


def kernel(x, edge_index, edge_weight, Wa1_c, wa2_c, Wa1_o, wa2_o, Wc, bc, Wo, bo):
    pad = E_PAD - E
    src3 = jnp.concatenate(
        [edge_index[0], jnp.zeros((pad,), jnp.int32)]).reshape(
            TILES, NBATCH, EB)
    dst3 = jnp.concatenate(
        [edge_index[1], jnp.zeros((pad,), jnp.int32)]).reshape(
            TILES, NBATCH, EB)
    w3 = jnp.concatenate(
        [edge_weight, jnp.zeros((pad,), jnp.float32)]).reshape(
            TILES, NBATCH, EB)
    ed = jnp.stack([src3, dst3], axis=2)
    z = jnp.zeros((ROWS_MAIN, CHUNK), jnp.float32)

    c_att, o_att = _att_pool(x, Wa1_c, wa2_c.reshape(H, 1),
                             Wa1_o, wa2_o.reshape(H, 1))

    h0 = jnp.concatenate([c_att, o_att], axis=0).reshape(NCHT * N, CHUNK)
    g1 = _conv(h0, ed, w3, z)
    g2 = _conv(g1, ed, w3, z)

    center, offset = _head(g2.reshape(NCHT, N, CHUNK), c_att, o_att,
                           Wc.reshape(NCH, CHUNK, H), bc.reshape(1, H),
                           Wo.reshape(NCH, CHUNK, H), bo.reshape(1, H))
    return (center, offset)


# speedup vs baseline: 3.4013x; 1.0946x over previous
import jax
import jax.numpy as jnp
from jax import lax
from jax.experimental import pallas as pl
from jax.experimental.pallas import tpu as pltpu
from jax.experimental.pallas import tpu_sc as plsc

N = 10000
L = 8
D = 768
H = 128
E = 160000

CHUNK = 128
NCH = D // CHUNK
NCHT = 2 * NCH
TILES = 16
EPT = E // TILES
EB = 128
NBATCH = 79
E_PAD = TILES * NBATCH * EB
ROWS_MAIN = 624
ROWS_TAIL = N - ROWS_MAIN * TILES

NB = 200
GRID = N // NB


def _att_body(x_ref, w1c_ref, w2c_ref, w1o_ref, w2o_ref, c_ref, o_ref):
    w1c = w1c_ref[...]
    w2c = w2c_ref[...]
    w1o = w1o_ref[...]
    w2o = w2o_ref[...]
    cols_c = []
    cols_o = []
    for l in range(L):
        xl = x_ref[:, l, :]
        hc = jnp.tanh(jnp.dot(xl, w1c, preferred_element_type=jnp.float32))
        cols_c.append(jnp.dot(hc, w2c, preferred_element_type=jnp.float32))
        ho = jnp.tanh(jnp.dot(xl, w1o, preferred_element_type=jnp.float32))
        cols_o.append(jnp.dot(ho, w2o, preferred_element_type=jnp.float32))
    a_c = jax.nn.softmax(jnp.concatenate(cols_c, axis=1), axis=1)
    a_o = jax.nn.softmax(jnp.concatenate(cols_o, axis=1), axis=1)
    pc = jnp.zeros((NB, D), jnp.float32)
    po = jnp.zeros((NB, D), jnp.float32)
    for l in range(L):
        xl = x_ref[:, l, :]
        pc = pc + a_c[:, l:l + 1] * xl
        po = po + a_o[:, l:l + 1] * xl
    for k in range(NCH):
        c_ref[k] = pc[:, k * CHUNK:(k + 1) * CHUNK]
        o_ref[k] = po[:, k * CHUNK:(k + 1) * CHUNK]


def _att_pool(x, w1c, w2c, w1o, w2o):
    out_sds = jax.ShapeDtypeStruct((NCH, N, CHUNK), jnp.float32)
    return pl.pallas_call(
        _att_body,
        grid=(GRID,),
        in_specs=[
            pl.BlockSpec((NB, L, D), lambda i: (i, 0, 0)),
            pl.BlockSpec((D, H), lambda i: (0, 0)),
            pl.BlockSpec((H, 1), lambda i: (0, 0)),
            pl.BlockSpec((D, H), lambda i: (0, 0)),
            pl.BlockSpec((H, 1), lambda i: (0, 0)),
        ],
        out_specs=[
            pl.BlockSpec((NCH, NB, CHUNK), lambda i: (0, i, 0)),
            pl.BlockSpec((NCH, NB, CHUNK), lambda i: (0, i, 0)),
        ],
        out_shape=[out_sds, out_sds],
    )(x, w1c, w2c, w1o, w2o)


def _conv_body(h_ref, ed_ref, w_ref, z_ref, out_ref,
               acc, eb0, eb1, gi0, gi1, dv0, dv1, rows0, rows1, w_v,
               se0, se1, sr0, sr1, ss0, ss1):
    cid = lax.axis_index("c")
    sid = lax.axis_index("s")

    pltpu.sync_copy(w_ref.at[sid], w_v)

    def gather(gi, rows, sem):
        pltpu.async_copy(h_ref.at[gi], rows, sem)

    def load_e(b, eb, sem):
        pltpu.async_copy(ed_ref.at[sid, b], eb, sem)

    def wait_e(b, eb, sem):
        pltpu.make_async_copy(ed_ref.at[sid, b], eb, sem).wait()

    def wait_r(gi, rows, sem):
        pltpu.make_async_copy(h_ref.at[gi], rows, sem).wait()

    def scat(rows, dv, sem):
        pltpu.async_copy(rows, acc.at[dv], sem, add=True)

    def wait_s(rows, dv, sem):
        pltpu.make_async_copy(rows, acc.at[dv], sem).wait()

    def mk_gidx(gi, eb, chunk):
        for q in range(EB // 16):
            gi[pl.ds(q * 16, 16)] = eb[0, pl.ds(q * 16, 16)] + chunk * N

    def scale(rows, b):
        def group_body(g, carry):
            wv = w_v[b, pl.ds(g * 16, 16)]
            for r0 in range(16):
                r = g * 16 + r0
                wr = wv[r0]
                for q in range(CHUNK // 16):
                    rows[r, pl.ds(q * 16, 16)] = (
                        rows[r, pl.ds(q * 16, 16)] * wr)
            return carry
        lax.fori_loop(0, EB // 16, group_body, 0, unroll=False)

    def copy_dst(dv, eb):
        for q in range(EB // 16):
            dv[pl.ds(q * 16, 16)] = eb[1, pl.ds(q * 16, 16)]

    def chunk_body(j, carry):
        chunk = cid * NCH + j

        pltpu.sync_copy(z_ref.at[pl.ds(0, ROWS_MAIN)],
                        acc.at[pl.ds(sid * ROWS_MAIN, ROWS_MAIN)])

        @pl.when(sid == TILES - 1)
        def _zero_tail():
            pltpu.sync_copy(z_ref.at[pl.ds(0, ROWS_TAIL)],
                            acc.at[pl.ds(ROWS_MAIN * TILES, ROWS_TAIL)])
        plsc.subcore_barrier()

        load_e(0, eb0, se0)
        wait_e(0, eb0, se0)
        mk_gidx(gi0, eb0, chunk)
        gather(gi0, rows0, sr0)
        load_e(1, eb1, se1)

        def pair_body(p, carry2):
            b = 2 * p
            wait_e(b + 1, eb1, se1)
            mk_gidx(gi1, eb1, chunk)

            @pl.when(p > 0)
            def _drain_s1():
                wait_s(rows1, dv1, ss1)
            gather(gi1, rows1, sr1)
            wait_r(gi0, rows0, sr0)
            scale(rows0, b)
            copy_dst(dv0, eb0)
            scat(rows0, dv0, ss0)
            load_e(b + 2, eb0, se0)
            wait_e(b + 2, eb0, se0)
            mk_gidx(gi0, eb0, chunk)
            wait_s(rows0, dv0, ss0)
            gather(gi0, rows0, sr0)
            wait_r(gi1, rows1, sr1)
            scale(rows1, b + 1)
            copy_dst(dv1, eb1)
            scat(rows1, dv1, ss1)
            load_e(jnp.minimum(b + 3, NBATCH - 1), eb1, se1)
            return carry2
        lax.fori_loop(0, (NBATCH - 1) // 2, pair_body, 0, unroll=False)

        wait_e(NBATCH - 1, eb1, se1)
        wait_s(rows1, dv1, ss1)
        wait_r(gi0, rows0, sr0)
        scale(rows0, NBATCH - 1)
        copy_dst(dv0, eb0)
        pltpu.sync_copy(rows0, acc.at[dv0], add=True)

        plsc.subcore_barrier()
        pltpu.sync_copy(
            acc.at[pl.ds(sid * ROWS_MAIN, ROWS_MAIN)],
            out_ref.at[pl.ds(chunk * N + sid * ROWS_MAIN, ROWS_MAIN)])

        @pl.when(sid == TILES - 1)
        def _read_tail():
            pltpu.sync_copy(
                acc.at[pl.ds(ROWS_MAIN * TILES, ROWS_TAIL)],
                out_ref.at[pl.ds(chunk * N + ROWS_MAIN * TILES, ROWS_TAIL)])
        return carry
    lax.fori_loop(0, NCH, chunk_body, 0, unroll=False)


def _conv(h_flat, ed, w3, z):
    mesh = plsc.VectorSubcoreMesh(core_axis_name="c", subcore_axis_name="s")
    f = pl.kernel(
        _conv_body,
        out_type=jax.ShapeDtypeStruct((NCHT * N, CHUNK), jnp.float32),
        mesh=mesh,
        scratch_types=[
            pltpu.VMEM_SHARED((N, CHUNK), jnp.float32),
            pltpu.VMEM((2, EB), jnp.int32),
            pltpu.VMEM((2, EB), jnp.int32),
            pltpu.VMEM((EB,), jnp.int32),
            pltpu.VMEM((EB,), jnp.int32),
            pltpu.VMEM((EB,), jnp.int32),
            pltpu.VMEM((EB,), jnp.int32),
            pltpu.VMEM((EB, CHUNK), jnp.float32),
            pltpu.VMEM((EB, CHUNK), jnp.float32),
            pltpu.VMEM((NBATCH, EB), jnp.float32),
            pltpu.SemaphoreType.DMA,
            pltpu.SemaphoreType.DMA,
            pltpu.SemaphoreType.DMA,
            pltpu.SemaphoreType.DMA,
            pltpu.SemaphoreType.DMA,
            pltpu.SemaphoreType.DMA,
        ],
    )
    return f(h_flat, ed, w3, z)


def _head_body(g_ref, c_ref, o_ref, wc_ref, bc_ref, wo_ref, bo_ref,
               ctr_ref, off_ref):
    acc_c = jnp.zeros((NB, H), jnp.float32)
    acc_o = jnp.zeros((NB, H), jnp.float32)
    for k in range(NCH):
        acc_c = acc_c + jnp.dot(g_ref[k] + c_ref[k], wc_ref[k],
                                preferred_element_type=jnp.float32)
        acc_o = acc_o + jnp.dot(g_ref[NCH + k] + o_ref[k], wo_ref[k],
                                preferred_element_type=jnp.float32)
    ctr_ref[...] = acc_c + bc_ref[...]
    off_ref[...] = jnp.maximum(acc_o + bo_ref[...], 0.0)


def _head(g2, c_att, o_att, wc, bc, wo, bo):
    out_sds = jax.ShapeDtypeStruct((N, H), jnp.float32)
    return pl.pallas_call(
        _head_body,
        grid=(GRID,),
        in_specs=[
            pl.BlockSpec((NCHT, NB, CHUNK), lambda i: (0, i, 0)),
            pl.BlockSpec((NCH, NB, CHUNK), lambda i: (0, i, 0)),
            pl.BlockSpec((NCH, NB, CHUNK), lambda i: (0, i, 0)),
            pl.BlockSpec((NCH, CHUNK, H), lambda i: (0, 0, 0)),
            pl.BlockSpec((1, H), lambda i: (0, 0)),
            pl.BlockSpec((NCH, CHUNK, H), lambda i: (0, 0, 0)),
            pl.BlockSpec((1, H), lambda i: (0, 0)),
        ],
        out_specs=[
            pl.BlockSpec((NB, H), lambda i: (i, 0)),
            pl.BlockSpec((NB, H), lambda i: (i, 0)),
        ],
        out_shape=[out_sds, out_sds],
    )(g2, c_att, o_att, wc, bc, wo, bo)


def kernel(x, edge_index, edge_weight, Wa1_c, wa2_c, Wa1_o, wa2_o, Wc, bc, Wo, bo):
    pad = E_PAD - E
    src3 = jnp.concatenate(
        [edge_index[0], jnp.zeros((pad,), jnp.int32)]).reshape(
            TILES, NBATCH, EB)
    dst3 = jnp.concatenate(
        [edge_index[1], jnp.zeros((pad,), jnp.int32)]).reshape(
            TILES, NBATCH, EB)
    w3 = jnp.concatenate(
        [edge_weight, jnp.zeros((pad,), jnp.float32)]).reshape(
            TILES, NBATCH, EB)
    ed = jnp.stack([src3, dst3], axis=2)
    z = jnp.zeros((ROWS_MAIN, CHUNK), jnp.float32)

    c_att, o_att = _att_pool(x, Wa1_c, wa2_c.reshape(H, 1),
                             Wa1_o, wa2_o.reshape(H, 1))

    h0 = jnp.concatenate([c_att, o_att], axis=0).reshape(NCHT * N, CHUNK)
    g1 = _conv(h0, ed, w3, z)
    g2 = _conv(g1, ed, w3, z)

    center, offset = _head(g2.reshape(NCHT, N, CHUNK), c_att, o_att,
                           Wc.reshape(NCH, CHUNK, H), bc.reshape(1, H),
                           Wo.reshape(NCH, CHUNK, H), bo.reshape(1, H))
    return (center, offset)
